# Initial kernel scaffold; baseline (speedup 1.0000x reference)
import jax
import jax.numpy as jnp
from jax.experimental import pallas as pl


def kernel(x, adj1, adj2, adj3, adj4, adj5, W1, b1, W2, b2, W3, b3, Wd, bd):
    raise NotImplementedError("write your pallas kernel here")


# speedup vs baseline: 1.3046x; 1.3046x over previous
import jax
import jax.numpy as jnp
from jax.experimental import pallas as pl
from jax.experimental.pallas import tpu as pltpu

N, F, H, C = 10000, 128, 128, 64
BM = 256
NB = (N + BM - 1) // BM
NP = NB * BM
F8 = jnp.float8_e4m3fn
QMAX = 256.0
BF16 = jnp.bfloat16
F32 = jnp.float32


def _support_kernel(x_ref, w_ref, o_ref):
    o_ref[...] = jnp.dot(
        x_ref[...].astype(BF16), w_ref[...], preferred_element_type=F32
    ).astype(F8)


def _layer1_kernel(a_ref, s_ref, b_ref, w_ref, q_ref, sc_ref, s2_ref):
    a = a_ref[...]
    amax = jnp.max(jnp.abs(a), axis=1, keepdims=True)
    amax = jnp.maximum(amax, 1e-30)
    aq = (a * (QMAX / amax)).astype(F8)
    q_ref[...] = aq
    sc_ref[...] = amax / QMAX
    z = jnp.dot(aq, s_ref[...], preferred_element_type=F32)
    z = z * (amax / QMAX) + b_ref[...]
    h = jnp.maximum(z, 0.0)
    s2_ref[...] = jnp.dot(
        h.astype(BF16), w_ref[...], preferred_element_type=F32
    ).astype(F8)


def _mid_kernel(q_ref, sc_ref, s_ref, b_ref, w_ref, o_ref):
    z = jnp.dot(q_ref[...], s_ref[...], preferred_element_type=F32)
    z = z * sc_ref[...] + b_ref[...]
    h = jnp.maximum(z, 0.0)
    o_ref[...] = jnp.dot(
        h.astype(BF16), w_ref[...], preferred_element_type=F32
    ).astype(F8)


def _last_kernel(q_ref, sc_ref, s_ref, b_ref, w_ref, bd_ref, o_ref):
    z = jnp.dot(q_ref[...], s_ref[...], preferred_element_type=F32)
    z = z * sc_ref[...] + b_ref[...]
    h = jnp.maximum(z, 0.0)
    o_ref[...] = (
        jnp.dot(h.astype(BF16), w_ref[...], preferred_element_type=F32)
        + bd_ref[...]
    )


_PARAMS = pltpu.CompilerParams(dimension_semantics=("parallel",))


def kernel(x, adj1, adj2, adj3, adj4, adj5, W1, b1, W2, b2, W3, b3, Wd, bd):
    W1b = W1.astype(BF16)
    W2b = W2.astype(BF16)
    W3b = W3.astype(BF16)
    Wdb = Wd.astype(BF16)
    b1r = b1.reshape(1, H)
    b2r = b2.reshape(1, H)
    b3r = b3.reshape(1, H)
    bdr = bd.reshape(1, C)

    s1 = pl.pallas_call(
        _support_kernel,
        out_shape=jax.ShapeDtypeStruct((N, H), F8),
    )(x, W1b)

    q, sc, s2 = pl.pallas_call(
        _layer1_kernel,
        grid=(NB,),
        in_specs=[
            pl.BlockSpec((BM, N), lambda i: (i, 0)),
            pl.BlockSpec((N, H), lambda i: (0, 0)),
            pl.BlockSpec((1, H), lambda i: (0, 0)),
            pl.BlockSpec((H, H), lambda i: (0, 0)),
        ],
        out_specs=[
            pl.BlockSpec((BM, N), lambda i: (i, 0)),
            pl.BlockSpec((BM, 1), lambda i: (i, 0)),
            pl.BlockSpec((BM, H), lambda i: (i, 0)),
        ],
        out_shape=[
            jax.ShapeDtypeStruct((NP, N), F8),
            jax.ShapeDtypeStruct((NP, 1), F32),
            jax.ShapeDtypeStruct((N, H), F8),
        ],
        compiler_params=_PARAMS,
    )(adj4, s1, b1r, W2b)

    s3 = pl.pallas_call(
        _mid_kernel,
        grid=(NB,),
        in_specs=[
            pl.BlockSpec((BM, N), lambda i: (i, 0)),
            pl.BlockSpec((BM, 1), lambda i: (i, 0)),
            pl.BlockSpec((N, H), lambda i: (0, 0)),
            pl.BlockSpec((1, H), lambda i: (0, 0)),
            pl.BlockSpec((H, H), lambda i: (0, 0)),
        ],
        out_specs=pl.BlockSpec((BM, H), lambda i: (i, 0)),
        out_shape=jax.ShapeDtypeStruct((N, H), F8),
        compiler_params=_PARAMS,
    )(q, sc, s2, b2r, W3b)

    out = pl.pallas_call(
        _last_kernel,
        grid=(NB,),
        in_specs=[
            pl.BlockSpec((BM, N), lambda i: (i, 0)),
            pl.BlockSpec((BM, 1), lambda i: (i, 0)),
            pl.BlockSpec((N, H), lambda i: (0, 0)),
            pl.BlockSpec((1, H), lambda i: (0, 0)),
            pl.BlockSpec((H, C), lambda i: (0, 0)),
            pl.BlockSpec((1, C), lambda i: (0, 0)),
        ],
        out_specs=pl.BlockSpec((BM, C), lambda i: (i, 0)),
        out_shape=jax.ShapeDtypeStruct((N, C), F32),
        compiler_params=_PARAMS,
    )(q, sc, s3, b3r, Wdb, bdr)

    return out


# speedup vs baseline: 1.4891x; 1.1414x over previous
import jax
import jax.numpy as jnp
from jax.experimental import pallas as pl
from jax.experimental.pallas import tpu as pltpu

N, F, H, C = 10000, 128, 128, 64
BM = 256
NB = (N + BM - 1) // BM
NP = NB * BM
BM2 = 1024
NB2 = NP // BM2
F8 = jnp.float8_e4m3fn
QMAX = 256.0
BF16 = jnp.bfloat16
F32 = jnp.float32


def _support_kernel(x_ref, w_ref, o_ref):
    o_ref[...] = jnp.dot(
        x_ref[...].astype(BF16), w_ref[...], preferred_element_type=F32
    ).astype(F8)


def _layer1_kernel(a_ref, s_ref, b_ref, w_ref, q_ref, sc_ref, s2_ref):
    a = a_ref[...]
    amax = jnp.max(jnp.abs(a), axis=1, keepdims=True)
    amax = jnp.maximum(amax, 1e-30)
    aq = (a * (QMAX / amax)).astype(F8)
    q_ref[...] = aq
    sc_ref[...] = amax / QMAX
    z = jnp.dot(aq, s_ref[...], preferred_element_type=F32)
    z = z * (amax / QMAX) + b_ref[...]
    h = jnp.maximum(z, 0.0)
    s2_ref[...] = jnp.dot(
        h.astype(BF16), w_ref[...], preferred_element_type=F32
    ).astype(F8)


def _mid_kernel(q_ref, sc_ref, s_ref, b_ref, w_ref, o_ref):
    z = jnp.dot(q_ref[...], s_ref[...], preferred_element_type=F32)
    z = z * sc_ref[...] + b_ref[...]
    h = jnp.maximum(z, 0.0)
    o_ref[...] = jnp.dot(
        h.astype(BF16), w_ref[...], preferred_element_type=F32
    ).astype(F8)


def _last_kernel(q_ref, sc_ref, s_ref, b_ref, w_ref, bd_ref, o_ref):
    z = jnp.dot(q_ref[...], s_ref[...], preferred_element_type=F32)
    z = z * sc_ref[...] + b_ref[...]
    h = jnp.maximum(z, 0.0)
    o_ref[...] = (
        jnp.dot(h.astype(BF16), w_ref[...], preferred_element_type=F32)
        + bd_ref[...]
    )


_PARAMS = pltpu.CompilerParams(dimension_semantics=("parallel",))


def kernel(x, adj1, adj2, adj3, adj4, adj5, W1, b1, W2, b2, W3, b3, Wd, bd):
    W1b = W1.astype(BF16)
    W2b = W2.astype(BF16)
    W3b = W3.astype(BF16)
    Wdb = Wd.astype(BF16)
    b1r = b1.reshape(1, H)
    b2r = b2.reshape(1, H)
    b3r = b3.reshape(1, H)
    bdr = bd.reshape(1, C)

    s1 = pl.pallas_call(
        _support_kernel,
        out_shape=jax.ShapeDtypeStruct((N, H), F8),
    )(x, W1b)

    q, sc, s2 = pl.pallas_call(
        _layer1_kernel,
        grid=(NB,),
        in_specs=[
            pl.BlockSpec((BM, N), lambda i: (i, 0)),
            pl.BlockSpec((N, H), lambda i: (0, 0)),
            pl.BlockSpec((1, H), lambda i: (0, 0)),
            pl.BlockSpec((H, H), lambda i: (0, 0)),
        ],
        out_specs=[
            pl.BlockSpec((BM, N), lambda i: (i, 0)),
            pl.BlockSpec((BM, 1), lambda i: (i, 0)),
            pl.BlockSpec((BM, H), lambda i: (i, 0)),
        ],
        out_shape=[
            jax.ShapeDtypeStruct((NP, N), F8),
            jax.ShapeDtypeStruct((NP, 1), F32),
            jax.ShapeDtypeStruct((N, H), F8),
        ],
        compiler_params=_PARAMS,
    )(adj4, s1, b1r, W2b)

    s3 = pl.pallas_call(
        _mid_kernel,
        grid=(NB2,),
        in_specs=[
            pl.BlockSpec((BM2, N), lambda i: (i, 0)),
            pl.BlockSpec((BM2, 1), lambda i: (i, 0)),
            pl.BlockSpec((N, H), lambda i: (0, 0)),
            pl.BlockSpec((1, H), lambda i: (0, 0)),
            pl.BlockSpec((H, H), lambda i: (0, 0)),
        ],
        out_specs=pl.BlockSpec((BM2, H), lambda i: (i, 0)),
        out_shape=jax.ShapeDtypeStruct((N, H), F8),
        compiler_params=_PARAMS,
    )(q, sc, s2, b2r, W3b)

    out = pl.pallas_call(
        _last_kernel,
        grid=(NB2,),
        in_specs=[
            pl.BlockSpec((BM2, N), lambda i: (i, 0)),
            pl.BlockSpec((BM2, 1), lambda i: (i, 0)),
            pl.BlockSpec((N, H), lambda i: (0, 0)),
            pl.BlockSpec((1, H), lambda i: (0, 0)),
            pl.BlockSpec((H, C), lambda i: (0, 0)),
            pl.BlockSpec((1, C), lambda i: (0, 0)),
        ],
        out_specs=pl.BlockSpec((BM2, C), lambda i: (i, 0)),
        out_shape=jax.ShapeDtypeStruct((N, C), F32),
        compiler_params=_PARAMS,
    )(q, sc, s3, b3r, Wdb, bdr)

    return out


# speedup vs baseline: 1.5046x; 1.0104x over previous
import jax
import jax.numpy as jnp
from jax.experimental import pallas as pl
from jax.experimental.pallas import tpu as pltpu

N, F, H, C = 10000, 128, 128, 64
BM = 512
NB = (N + BM - 1) // BM
NP = NB * BM
BM2 = 1024
NB2 = NP // BM2
F8 = jnp.float8_e4m3fn
QMAX = 256.0
BF16 = jnp.bfloat16
F32 = jnp.float32


def _support_kernel(x_ref, w_ref, o_ref):
    o_ref[...] = jnp.dot(
        x_ref[...].astype(BF16), w_ref[...], preferred_element_type=F32
    ).astype(F8)


def _layer1_kernel(a_ref, s_ref, b_ref, w_ref, q_ref, sc_ref, s2_ref):
    a = a_ref[...]
    amax = jnp.max(jnp.abs(a), axis=1, keepdims=True)
    amax = jnp.maximum(amax, 1e-30)
    aq = (a * (QMAX / amax)).astype(F8)
    q_ref[...] = aq
    sc_ref[...] = amax / QMAX
    z = jnp.dot(aq, s_ref[...], preferred_element_type=F32)
    z = z * (amax / QMAX) + b_ref[...]
    h = jnp.maximum(z, 0.0)
    s2_ref[...] = jnp.dot(
        h.astype(BF16), w_ref[...], preferred_element_type=F32
    ).astype(F8)


def _mid_kernel(q_ref, sc_ref, s_ref, b_ref, w_ref, o_ref):
    z = jnp.dot(q_ref[...], s_ref[...], preferred_element_type=F32)
    z = z * sc_ref[...] + b_ref[...]
    h = jnp.maximum(z, 0.0)
    o_ref[...] = jnp.dot(
        h.astype(BF16), w_ref[...], preferred_element_type=F32
    ).astype(F8)


def _last_kernel(q_ref, sc_ref, s_ref, b_ref, w_ref, bd_ref, o_ref):
    z = jnp.dot(q_ref[...], s_ref[...], preferred_element_type=F32)
    z = z * sc_ref[...] + b_ref[...]
    h = jnp.maximum(z, 0.0)
    o_ref[...] = (
        jnp.dot(h.astype(BF16), w_ref[...], preferred_element_type=F32)
        + bd_ref[...]
    )


_PARAMS = pltpu.CompilerParams(dimension_semantics=("parallel",))


def kernel(x, adj1, adj2, adj3, adj4, adj5, W1, b1, W2, b2, W3, b3, Wd, bd):
    W1b = W1.astype(BF16)
    W2b = W2.astype(BF16)
    W3b = W3.astype(BF16)
    Wdb = Wd.astype(BF16)
    b1r = b1.reshape(1, H)
    b2r = b2.reshape(1, H)
    b3r = b3.reshape(1, H)
    bdr = bd.reshape(1, C)

    s1 = pl.pallas_call(
        _support_kernel,
        out_shape=jax.ShapeDtypeStruct((N, H), F8),
    )(x, W1b)

    q, sc, s2 = pl.pallas_call(
        _layer1_kernel,
        grid=(NB,),
        in_specs=[
            pl.BlockSpec((BM, N), lambda i: (i, 0)),
            pl.BlockSpec((N, H), lambda i: (0, 0)),
            pl.BlockSpec((1, H), lambda i: (0, 0)),
            pl.BlockSpec((H, H), lambda i: (0, 0)),
        ],
        out_specs=[
            pl.BlockSpec((BM, N), lambda i: (i, 0)),
            pl.BlockSpec((BM, 1), lambda i: (i, 0)),
            pl.BlockSpec((BM, H), lambda i: (i, 0)),
        ],
        out_shape=[
            jax.ShapeDtypeStruct((NP, N), F8),
            jax.ShapeDtypeStruct((NP, 1), F32),
            jax.ShapeDtypeStruct((N, H), F8),
        ],
        compiler_params=_PARAMS,
    )(adj4, s1, b1r, W2b)

    s3 = pl.pallas_call(
        _mid_kernel,
        grid=(NB2,),
        in_specs=[
            pl.BlockSpec((BM2, N), lambda i: (i, 0)),
            pl.BlockSpec((BM2, 1), lambda i: (i, 0)),
            pl.BlockSpec((N, H), lambda i: (0, 0)),
            pl.BlockSpec((1, H), lambda i: (0, 0)),
            pl.BlockSpec((H, H), lambda i: (0, 0)),
        ],
        out_specs=pl.BlockSpec((BM2, H), lambda i: (i, 0)),
        out_shape=jax.ShapeDtypeStruct((N, H), F8),
        compiler_params=_PARAMS,
    )(q, sc, s2, b2r, W3b)

    out = pl.pallas_call(
        _last_kernel,
        grid=(NB2,),
        in_specs=[
            pl.BlockSpec((BM2, N), lambda i: (i, 0)),
            pl.BlockSpec((BM2, 1), lambda i: (i, 0)),
            pl.BlockSpec((N, H), lambda i: (0, 0)),
            pl.BlockSpec((1, H), lambda i: (0, 0)),
            pl.BlockSpec((H, C), lambda i: (0, 0)),
            pl.BlockSpec((1, C), lambda i: (0, 0)),
        ],
        out_specs=pl.BlockSpec((BM2, C), lambda i: (i, 0)),
        out_shape=jax.ShapeDtypeStruct((N, C), F32),
        compiler_params=_PARAMS,
    )(q, sc, s3, b3r, Wdb, bdr)

    return out


# speedup vs baseline: 1.5048x; 1.0001x over previous
import jax
import jax.numpy as jnp
from jax.experimental import pallas as pl
from jax.experimental.pallas import tpu as pltpu

N, F, H, C = 10000, 128, 128, 64
BM = 512
NB = (N + BM - 1) // BM
NP = NB * BM
BM2 = 1280
NB2 = NP // BM2
F8 = jnp.float8_e4m3fn
QMAX = 256.0
BF16 = jnp.bfloat16
F32 = jnp.float32


def _support_kernel(x_ref, w_ref, o_ref):
    o_ref[...] = jnp.dot(
        x_ref[...].astype(BF16), w_ref[...], preferred_element_type=F32
    ).astype(F8)


def _layer1_kernel(a_ref, s_ref, b_ref, w_ref, q_ref, sc_ref, s2_ref):
    a = a_ref[...]
    amax = jnp.max(jnp.abs(a), axis=1, keepdims=True)
    amax = jnp.maximum(amax, 1e-30)
    aq = (a * (QMAX / amax)).astype(F8)
    q_ref[...] = aq
    sc_ref[...] = amax / QMAX
    z = jnp.dot(aq, s_ref[...], preferred_element_type=F32)
    z = z * (amax / QMAX) + b_ref[...]
    h = jnp.maximum(z, 0.0)
    s2_ref[...] = jnp.dot(
        h.astype(BF16), w_ref[...], preferred_element_type=F32
    ).astype(F8)


def _mid_kernel(q_ref, sc_ref, s_ref, b_ref, w_ref, o_ref):
    z = jnp.dot(q_ref[...], s_ref[...], preferred_element_type=F32)
    z = z * sc_ref[...] + b_ref[...]
    h = jnp.maximum(z, 0.0)
    o_ref[...] = jnp.dot(
        h.astype(BF16), w_ref[...], preferred_element_type=F32
    ).astype(F8)


def _last_kernel(q_ref, sc_ref, s_ref, b_ref, w_ref, bd_ref, o_ref):
    z = jnp.dot(q_ref[...], s_ref[...], preferred_element_type=F32)
    z = z * sc_ref[...] + b_ref[...]
    h = jnp.maximum(z, 0.0)
    o_ref[...] = (
        jnp.dot(h.astype(BF16), w_ref[...], preferred_element_type=F32)
        + bd_ref[...]
    )


_PARAMS = pltpu.CompilerParams(dimension_semantics=("parallel",))


def kernel(x, adj1, adj2, adj3, adj4, adj5, W1, b1, W2, b2, W3, b3, Wd, bd):
    W1b = W1.astype(BF16)
    W2b = W2.astype(BF16)
    W3b = W3.astype(BF16)
    Wdb = Wd.astype(BF16)
    b1r = b1.reshape(1, H)
    b2r = b2.reshape(1, H)
    b3r = b3.reshape(1, H)
    bdr = bd.reshape(1, C)

    s1 = pl.pallas_call(
        _support_kernel,
        out_shape=jax.ShapeDtypeStruct((N, H), F8),
    )(x, W1b)

    q, sc, s2 = pl.pallas_call(
        _layer1_kernel,
        grid=(NB,),
        in_specs=[
            pl.BlockSpec((BM, N), lambda i: (i, 0)),
            pl.BlockSpec((N, H), lambda i: (0, 0)),
            pl.BlockSpec((1, H), lambda i: (0, 0)),
            pl.BlockSpec((H, H), lambda i: (0, 0)),
        ],
        out_specs=[
            pl.BlockSpec((BM, N), lambda i: (i, 0)),
            pl.BlockSpec((BM, 1), lambda i: (i, 0)),
            pl.BlockSpec((BM, H), lambda i: (i, 0)),
        ],
        out_shape=[
            jax.ShapeDtypeStruct((NP, N), F8),
            jax.ShapeDtypeStruct((NP, 1), F32),
            jax.ShapeDtypeStruct((N, H), F8),
        ],
        compiler_params=_PARAMS,
    )(adj4, s1, b1r, W2b)

    s3 = pl.pallas_call(
        _mid_kernel,
        grid=(NB2,),
        in_specs=[
            pl.BlockSpec((BM2, N), lambda i: (i, 0)),
            pl.BlockSpec((BM2, 1), lambda i: (i, 0)),
            pl.BlockSpec((N, H), lambda i: (0, 0)),
            pl.BlockSpec((1, H), lambda i: (0, 0)),
            pl.BlockSpec((H, H), lambda i: (0, 0)),
        ],
        out_specs=pl.BlockSpec((BM2, H), lambda i: (i, 0)),
        out_shape=jax.ShapeDtypeStruct((N, H), F8),
        compiler_params=_PARAMS,
    )(q, sc, s2, b2r, W3b)

    out = pl.pallas_call(
        _last_kernel,
        grid=(NB2,),
        in_specs=[
            pl.BlockSpec((BM2, N), lambda i: (i, 0)),
            pl.BlockSpec((BM2, 1), lambda i: (i, 0)),
            pl.BlockSpec((N, H), lambda i: (0, 0)),
            pl.BlockSpec((1, H), lambda i: (0, 0)),
            pl.BlockSpec((H, C), lambda i: (0, 0)),
            pl.BlockSpec((1, C), lambda i: (0, 0)),
        ],
        out_specs=pl.BlockSpec((BM2, C), lambda i: (i, 0)),
        out_shape=jax.ShapeDtypeStruct((N, C), F32),
        compiler_params=_PARAMS,
    )(q, sc, s3, b3r, Wdb, bdr)

    return out


# speedup vs baseline: 1.5451x; 1.0268x over previous
import jax
import jax.numpy as jnp
from jax.experimental import pallas as pl
from jax.experimental.pallas import tpu as pltpu

N, F, H, C = 10000, 128, 128, 64
BM = 512
NB = (N + BM - 1) // BM
NP = NB * BM
BM2 = 1280
NB2 = NP // BM2
F8 = jnp.float8_e4m3fn
QMAX = 256.0
BF16 = jnp.bfloat16
F32 = jnp.float32


def _support_kernel(x_ref, w_ref, o_ref):
    o_ref[...] = jnp.dot(
        x_ref[...].astype(BF16), w_ref[...], preferred_element_type=F32
    ).astype(F8)


def _layer1_kernel(a_ref, s_ref, b_ref, w_ref, q_ref, sc_ref, s2_ref):
    a = a_ref[...]
    amax = jnp.max(jnp.abs(a), axis=1, keepdims=True)
    amax = jnp.maximum(amax, 1e-30)
    aq = (a * (QMAX / amax)).astype(F8)
    q_ref[...] = aq
    sc_ref[...] = amax / QMAX
    z = jnp.dot(aq, s_ref[...], preferred_element_type=F32)
    z = z * (amax / QMAX) + b_ref[...]
    h = jnp.maximum(z, 0.0)
    s2_ref[...] = jnp.dot(
        h.astype(BF16), w_ref[...], preferred_element_type=F32
    ).astype(F8)


LAST = N - (NB2 - 1) * BM2


def _merged_kernel(q_ref, sc_ref, s2_ref, b_ref, w3_ref, wd_ref, bd_ref,
                   o_ref, s3_ref):
    p = pl.program_id(0)
    i = pl.program_id(1)

    @pl.when(p == 0)
    def _phase0():
        z = jnp.dot(q_ref[...], s2_ref[...], preferred_element_type=F32)
        z = z * sc_ref[...] + b_ref[0]
        h = jnp.maximum(z, 0.0)
        v = jnp.dot(
            h.astype(BF16), w3_ref[...], preferred_element_type=F32
        ).astype(F8)

        @pl.when(i < NB2 - 1)
        def _():
            s3_ref[pl.ds(i * BM2, BM2), :] = v

        @pl.when(i == NB2 - 1)
        def _():
            s3_ref[pl.ds(N - LAST, LAST), :] = v[:LAST]

    @pl.when(p == 1)
    def _phase1():
        z = jnp.dot(q_ref[...], s3_ref[...], preferred_element_type=F32)
        z = z * sc_ref[...] + b_ref[0]
        h = jnp.maximum(z, 0.0)
        o_ref[...] = (
            jnp.dot(h.astype(BF16), wd_ref[...], preferred_element_type=F32)
            + bd_ref[...]
        )


_PARAMS = pltpu.CompilerParams(dimension_semantics=("parallel",))


def kernel(x, adj1, adj2, adj3, adj4, adj5, W1, b1, W2, b2, W3, b3, Wd, bd):
    W1b = W1.astype(BF16)
    W2b = W2.astype(BF16)
    W3b = W3.astype(BF16)
    Wdb = Wd.astype(BF16)
    b1r = b1.reshape(1, H)
    b2r = b2.reshape(1, H)
    b3r = b3.reshape(1, H)
    bdr = bd.reshape(1, C)

    s1 = pl.pallas_call(
        _support_kernel,
        out_shape=jax.ShapeDtypeStruct((N, H), F8),
    )(x, W1b)

    q, sc, s2 = pl.pallas_call(
        _layer1_kernel,
        grid=(NB,),
        in_specs=[
            pl.BlockSpec((BM, N), lambda i: (i, 0)),
            pl.BlockSpec((N, H), lambda i: (0, 0)),
            pl.BlockSpec((1, H), lambda i: (0, 0)),
            pl.BlockSpec((H, H), lambda i: (0, 0)),
        ],
        out_specs=[
            pl.BlockSpec((BM, N), lambda i: (i, 0)),
            pl.BlockSpec((BM, 1), lambda i: (i, 0)),
            pl.BlockSpec((BM, H), lambda i: (i, 0)),
        ],
        out_shape=[
            jax.ShapeDtypeStruct((NP, N), F8),
            jax.ShapeDtypeStruct((NP, 1), F32),
            jax.ShapeDtypeStruct((N, H), F8),
        ],
        compiler_params=_PARAMS,
    )(adj4, s1, b1r, W2b)

    b23 = jnp.stack([b2r, b3r], axis=0)

    out = pl.pallas_call(
        _merged_kernel,
        grid=(2, NB2),
        in_specs=[
            pl.BlockSpec((BM2, N), lambda p, i: (i, 0)),
            pl.BlockSpec((BM2, 1), lambda p, i: (i, 0)),
            pl.BlockSpec((N, H), lambda p, i: (0, 0)),
            pl.BlockSpec((1, 1, H), lambda p, i: (p, 0, 0)),
            pl.BlockSpec((H, H), lambda p, i: (0, 0)),
            pl.BlockSpec((H, C), lambda p, i: (0, 0)),
            pl.BlockSpec((1, C), lambda p, i: (0, 0)),
        ],
        out_specs=pl.BlockSpec((BM2, C), lambda p, i: (i, 0)),
        out_shape=jax.ShapeDtypeStruct((N, C), F32),
        scratch_shapes=[pltpu.VMEM((N, H), F8)],
        compiler_params=pltpu.CompilerParams(
            dimension_semantics=("arbitrary", "arbitrary")
        ),
    )(q, sc, s2, b23, W3b, Wdb, bdr)

    return out
